# Initial kernel scaffold; baseline (speedup 1.0000x reference)
import jax
import jax.numpy as jnp
from jax.experimental import pallas as pl


def kernel(input, Wg, W1, b1, W2, b2):
    raise NotImplementedError("write your pallas kernel here")


# speedup vs baseline: 2.0876x; 2.0876x over previous
import functools
import math

import jax
import jax.numpy as jnp
from jax import lax
from jax.experimental import pallas as pl
from jax.experimental.pallas import tpu as pltpu

_E = 8
_TOP_K = 2
_D = 1024
_F = 4096
_CAP_FACTOR = 1.25

_FT = 512


def _mlp_body(buf_ref, w1_ref, b1_ref, w2_ref, b2_ref, out_ref):
    f = pl.program_id(1)
    x = buf_ref[0]
    h = jnp.dot(x, w1_ref[0], preferred_element_type=jnp.float32)
    h = h + b1_ref[0, 0]
    h = 0.5 * h * (1.0 + lax.erf(h * (1.0 / math.sqrt(2.0))))
    p = jnp.dot(h, w2_ref[0], preferred_element_type=jnp.float32)

    @pl.when(f == 0)
    def _():
        out_ref[0] = p + b2_ref[0, 0]

    @pl.when(f > 0)
    def _():
        out_ref[0] += p


def _expert_mlp(buf, W1, b1, W2, b2, C):
    nf = _F // _FT
    return pl.pallas_call(
        _mlp_body,
        grid=(_E, nf),
        in_specs=[
            pl.BlockSpec((1, C, _D), lambda e, f: (e, 0, 0)),
            pl.BlockSpec((1, _D, _FT), lambda e, f: (e, 0, f)),
            pl.BlockSpec((1, 1, _FT), lambda e, f: (e, 0, f)),
            pl.BlockSpec((1, _FT, _D), lambda e, f: (e, f, 0)),
            pl.BlockSpec((1, 1, _D), lambda e, f: (e, 0, 0)),
        ],
        out_specs=pl.BlockSpec((1, C, _D), lambda e, f: (e, 0, 0)),
        out_shape=jax.ShapeDtypeStruct((_E, C, _D), jnp.float32),
        compiler_params=pltpu.CompilerParams(
            dimension_semantics=("parallel", "arbitrary"),
        ),
    )(buf, W1, b1[:, None, :], W2, b2[:, None, :])


def kernel(input, Wg, W1, b1, W2, b2):
    B, S, Dm = input.shape
    T = B * S
    xf = input.reshape(T, Dm)
    C = int(_CAP_FACTOR * T * _TOP_K / _E)
    logits = xf @ Wg
    gates = jax.nn.softmax(logits, axis=-1)
    topv, topi = jax.lax.top_k(gates, _TOP_K)
    denom = jnp.sum(topv, axis=-1, keepdims=True) + 1e-9
    topw = topv / denom
    e_flat = topi.T.reshape(-1)
    w_flat = topw.T.reshape(-1)
    oh = jax.nn.one_hot(e_flat, _E, dtype=jnp.int32)
    pos_in_e = jnp.cumsum(oh, axis=0) - oh
    pos = jnp.sum(pos_in_e * oh, axis=1)
    keep = pos < C
    pos_c = jnp.where(keep, pos, 0)
    keep_f = keep.astype(xf.dtype)
    x_rep = jnp.tile(xf, (_TOP_K, 1))
    vals = x_rep * keep_f[:, None]
    buf = jnp.zeros((_E, C, Dm), dtype=xf.dtype).at[e_flat, pos_c].add(vals)
    eo = _expert_mlp(buf, W1, b1, W2, b2, C)
    gathered = eo[e_flat, pos_c]
    gathered = gathered * (keep_f * w_flat)[:, None]
    y = gathered.reshape(_TOP_K, T, Dm).sum(axis=0)
    out = y.reshape(B, S, Dm)
    aux = jnp.zeros((Dm,), dtype=input.dtype)
    return (out, aux)
